# baseline (speedup 1.0000x reference)
import functools

import jax
import jax.numpy as jnp
from jax import lax
from jax.experimental import pallas as pl
from jax.experimental.pallas import tpu as pltpu
from jax.experimental.pallas import tpu_sc as plsc

T, D, F, E = 2048, 768, 3072, 8
EP = 128
BLK = 128
NB = T // BLK + E
PMAX = NB * BLK
NC, NS = 2, 16
NW = NC * NS
RP = PMAX // NW
TPW = T // NW


def _router_body(x_ref, wr_ref, pos_ref, gate_ref, bmeta_ref):
    x = x_ref[...]
    logits = jnp.dot(x, wr_ref[...], preferred_element_type=jnp.float32)
    lane = lax.broadcasted_iota(jnp.int32, (T, EP), 1)
    valid = lane < E
    logits = jnp.where(valid, logits, jnp.float32(-1e30))
    m = jnp.max(logits, axis=1, keepdims=True)
    p = jnp.where(valid, jnp.exp(logits - m), 0.0)
    probs = p / jnp.sum(p, axis=1, keepdims=True)
    gate = jnp.max(probs, axis=1, keepdims=True)
    cand = jnp.where((probs >= gate) & valid, lane, EP)
    expert = jnp.min(cand, axis=1, keepdims=True)
    onehot = (lane == expert).astype(jnp.float32)
    counts = jnp.sum(onehot, axis=0, keepdims=True)
    r_i = lax.broadcasted_iota(jnp.int32, (T, T), 0)
    c_i = lax.broadcasted_iota(jnp.int32, (T, T), 1)
    ltri = (c_i < r_i).astype(jnp.float32)
    ranks_all = jnp.dot(ltri, onehot, preferred_element_type=jnp.float32)
    rank = jnp.sum(ranks_all * onehot, axis=1, keepdims=True)
    nb_e = (counts.astype(jnp.int32) + (BLK - 1)) // BLK
    e_r = lax.broadcasted_iota(jnp.int32, (EP, EP), 0)
    e_c = lax.broadcasted_iota(jnp.int32, (EP, EP), 1)
    ustrict = (e_r < e_c).astype(jnp.float32)
    nb8 = jnp.broadcast_to(nb_e.astype(jnp.float32), (8, EP))
    bstart = jnp.dot(nb8, ustrict,
                     preferred_element_type=jnp.float32)[0:1].astype(jnp.int32)
    pstart = bstart * BLK
    pos = (jnp.sum(onehot * pstart.astype(jnp.float32), axis=1, keepdims=True)
           + rank).astype(jnp.int32)
    pos_ref[...] = pos
    gate_ref[...] = gate
    nbtot = jnp.sum(nb_e, axis=1, keepdims=True)
    lane1 = lax.broadcasted_iota(jnp.int32, (1, EP), 1)
    last_e = jnp.max(jnp.where(nb_e > 0, lane1, 0), axis=1, keepdims=True)
    j_r = lax.broadcasted_iota(jnp.int32, (32, EP), 0)
    e_l = lax.broadcasted_iota(jnp.int32, (32, EP), 1)
    covered = ((j_r >= bstart) & (e_l < E)).astype(jnp.int32)
    be = jnp.sum(covered, axis=1, keepdims=True) - 1
    j_c = lax.broadcasted_iota(jnp.int32, (32, 1), 0)
    be = jnp.where(j_c < nbtot, be, last_e)
    bmeta_ref[...] = jnp.where(j_c == 31, nbtot, be)


def _router_call(x, wr_p):
    return pl.pallas_call(
        _router_body,
        out_shape=(
            jax.ShapeDtypeStruct((T, 1), jnp.int32),
            jax.ShapeDtypeStruct((T, 1), jnp.float32),
            jax.ShapeDtypeStruct((32, 1), jnp.int32),
        ),
    )(x, wr_p)


TG = T // NS
ZG = PMAX // NS


def _dispatch_body(pos_hbm, gate_hbm, gpad_hbm, idxp_hbm,
                   posg_v, gatew_v, tid_v, zf_v, zi_v, gpw_v, ipw_v,
                   gate_sh, idx_sh):
    c = lax.axis_index("c")
    s = lax.axis_index("s")
    gb = s * TG
    zb = s * ZG
    pltpu.sync_copy(pos_hbm.at[pl.ds(gb, TG)], posg_v)

    @pl.when(c == 0)
    def _():
        pltpu.sync_copy(gate_hbm.at[pl.ds(gb, TG)], gatew_v)
        zeros16 = jnp.zeros((16,), jnp.float32)

        def _init(i, carry):
            zf_v[pl.ds(i * 16, 16)] = zeros16
            return carry

        lax.fori_loop(0, ZG // 16, _init, 0)
        pltpu.sync_copy(zf_v, gate_sh.at[pl.ds(zb, ZG)])
        plsc.subcore_barrier()
        pltpu.sync_copy(gatew_v, gate_sh.at[posg_v], add=True)
        plsc.subcore_barrier()
        pltpu.sync_copy(gate_sh.at[pl.ds(zb, ZG)], gpw_v)
        pltpu.sync_copy(gpw_v, gpad_hbm.at[pl.ds(zb, ZG)])

    @pl.when(c == 1)
    def _():
        izeros16 = jnp.zeros((16,), jnp.int32)
        lane = lax.broadcasted_iota(jnp.int32, (16,), 0)

        def _tid(i, carry):
            tid_v[pl.ds(i * 16, 16)] = lane + (gb + i * 16 + 1)
            return carry

        lax.fori_loop(0, TG // 16, _tid, 0)

        def _initi(i, carry):
            zi_v[pl.ds(i * 16, 16)] = izeros16
            return carry

        lax.fori_loop(0, ZG // 16, _initi, 0)
        pltpu.sync_copy(zi_v, idx_sh.at[pl.ds(zb, ZG)])
        plsc.subcore_barrier()
        pltpu.sync_copy(tid_v, idx_sh.at[posg_v], add=True)
        plsc.subcore_barrier()
        pltpu.sync_copy(idx_sh.at[pl.ds(zb, ZG)], ipw_v)
        pltpu.sync_copy(ipw_v, idxp_hbm.at[pl.ds(zb, ZG)])


def _ffn_body(bm_ref, x_ref, ia_ref, w1_ref, w2_ref, g_ref, out_ref):
    j = pl.program_id(0)
    nbtot = bm_ref[31]

    @pl.when(j < nbtot)
    def _():
        tok = ia_ref[0] - 1
        it = lax.broadcasted_iota(jnp.int32, (BLK, T), 1)
        sel = (it == tok).astype(jnp.float32)
        xb = jnp.dot(sel, x_ref[...], preferred_element_type=jnp.float32)
        h = jax.nn.gelu(jnp.dot(xb, w1_ref[0],
                                preferred_element_type=jnp.float32))
        o = jnp.dot(h, w2_ref[0], preferred_element_type=jnp.float32)
        out_ref[...] = o * g_ref[0]


def _ffn_call(bmeta, x, idx3a, w1, w2, gpad3):
    grid_spec = pltpu.PrefetchScalarGridSpec(
        num_scalar_prefetch=1,
        grid=(NB,),
        in_specs=[
            pl.BlockSpec((T, D), lambda j, bm: (0, 0)),
            pl.BlockSpec((1, BLK, 1),
                         lambda j, bm: (jnp.minimum(j, bm[31] - 1), 0, 0)),
            pl.BlockSpec((1, D, F), lambda j, bm: (bm[j], 0, 0)),
            pl.BlockSpec((1, F, D), lambda j, bm: (bm[j], 0, 0)),
            pl.BlockSpec((1, BLK, 1),
                         lambda j, bm: (jnp.minimum(j, bm[31] - 1), 0, 0)),
        ],
        out_specs=pl.BlockSpec((BLK, D),
                               lambda j, bm: (jnp.minimum(j, bm[31] - 1), 0)),
    )
    return pl.pallas_call(
        _ffn_body,
        grid_spec=grid_spec,
        out_shape=jax.ShapeDtypeStruct((PMAX, D), jnp.float32),
        compiler_params=pltpu.CompilerParams(
            dimension_semantics=("arbitrary",)),
    )(bmeta, x, idx3a, w1, w2, gpad3)


def _combine_body(os_hbm, pos_hbm, out_hbm, posw_v, rows_v, sem):
    c = lax.axis_index("c")
    s = lax.axis_index("s")
    wid = s * NC + c
    base = wid * TPW
    pltpu.sync_copy(pos_hbm.at[pl.ds(base, TPW)], posw_v)
    pltpu.async_copy(os_hbm.at[posw_v], rows_v, sem).wait()
    pltpu.sync_copy(rows_v, out_hbm.at[pl.ds(base, TPW)])


@functools.cache
def _sc_kernels():
    mesh = plsc.VectorSubcoreMesh(core_axis_name="c", subcore_axis_name="s")
    dispatch = pl.kernel(
        _dispatch_body,
        mesh=mesh,
        out_type=[
            jax.ShapeDtypeStruct((PMAX,), jnp.float32),
            jax.ShapeDtypeStruct((PMAX,), jnp.int32),
        ],
        scratch_types=[
            pltpu.VMEM((TG,), jnp.int32),
            pltpu.VMEM((TG,), jnp.float32),
            pltpu.VMEM((TG,), jnp.int32),
            pltpu.VMEM((ZG,), jnp.float32),
            pltpu.VMEM((ZG,), jnp.int32),
            pltpu.VMEM((ZG,), jnp.float32),
            pltpu.VMEM((ZG,), jnp.int32),
            pltpu.VMEM_SHARED((PMAX,), jnp.float32),
            pltpu.VMEM_SHARED((PMAX,), jnp.int32),
        ],
        compiler_params=pltpu.CompilerParams(needs_layout_passes=False),
    )
    combine = pl.kernel(
        _combine_body,
        mesh=mesh,
        out_type=jax.ShapeDtypeStruct((T, D), jnp.float32),
        scratch_types=[
            pltpu.VMEM((TPW,), jnp.int32),
            pltpu.VMEM((TPW, D), jnp.float32),
            pltpu.SemaphoreType.DMA,
        ],
        compiler_params=pltpu.CompilerParams(needs_layout_passes=False),
    )
    return dispatch, combine


def kernel(x, w_router, w1, w2):
    wr_p = jnp.pad(w_router, ((0, 0), (0, EP - E)))
    pos2, gate2, bmeta2 = _router_call(x, wr_p)
    pos = pos2.reshape(T)
    gate = gate2.reshape(T)
    bmeta = bmeta2.reshape(32)
    dispatch, combine = _sc_kernels()
    gpad, idxp = dispatch(pos, gate)
    gpad3 = gpad.reshape(NB, BLK, 1)
    idx3a = idxp.reshape(NB, BLK, 1)
    out_sorted = _ffn_call(bmeta, x, idx3a, w1, w2, gpad3)
    return combine(out_sorted, pos)


# speedup vs baseline: 1.0481x; 1.0481x over previous
import functools

import jax
import jax.numpy as jnp
from jax import lax
from jax.experimental import pallas as pl
from jax.experimental.pallas import tpu as pltpu
from jax.experimental.pallas import tpu_sc as plsc

T, D, F, E = 2048, 768, 3072, 8
EP = 128
BLK = 128
NB = T // BLK + E
PMAX = NB * BLK
NC, NS = 2, 16
NW = NC * NS
RP = PMAX // NW
TPW = T // NW


def _router_body(x_ref, wr_ref, pos_ref, gate_ref, bmeta_ref):
    x = x_ref[...]
    logits = jnp.dot(x, wr_ref[...], preferred_element_type=jnp.float32)
    lane = lax.broadcasted_iota(jnp.int32, (T, EP), 1)
    valid = lane < E
    logits = jnp.where(valid, logits, jnp.float32(-1e30))
    m = jnp.max(logits, axis=1, keepdims=True)
    p = jnp.where(valid, jnp.exp(logits - m), 0.0)
    probs = p / jnp.sum(p, axis=1, keepdims=True)
    gate = jnp.max(probs, axis=1, keepdims=True)
    cand = jnp.where((probs >= gate) & valid, lane, EP)
    expert = jnp.min(cand, axis=1, keepdims=True)
    onehot = (lane == expert).astype(jnp.float32)
    counts = jnp.sum(onehot, axis=0, keepdims=True)
    r_i = lax.broadcasted_iota(jnp.int32, (T, T), 0)
    c_i = lax.broadcasted_iota(jnp.int32, (T, T), 1)
    ltri = (c_i < r_i).astype(jnp.float32)
    ranks_all = jnp.dot(ltri, onehot, preferred_element_type=jnp.float32)
    rank = jnp.sum(ranks_all * onehot, axis=1, keepdims=True)
    nb_e = (counts.astype(jnp.int32) + (BLK - 1)) // BLK
    e_r = lax.broadcasted_iota(jnp.int32, (EP, EP), 0)
    e_c = lax.broadcasted_iota(jnp.int32, (EP, EP), 1)
    ustrict = (e_r < e_c).astype(jnp.float32)
    nb8 = jnp.broadcast_to(nb_e.astype(jnp.float32), (8, EP))
    bstart = jnp.dot(nb8, ustrict,
                     preferred_element_type=jnp.float32)[0:1].astype(jnp.int32)
    pstart = bstart * BLK
    pos = (jnp.sum(onehot * pstart.astype(jnp.float32), axis=1, keepdims=True)
           + rank).astype(jnp.int32)
    pos_ref[...] = pos
    gate_ref[...] = gate
    nbtot = jnp.sum(nb_e, axis=1, keepdims=True)
    lane1 = lax.broadcasted_iota(jnp.int32, (1, EP), 1)
    last_e = jnp.max(jnp.where(nb_e > 0, lane1, 0), axis=1, keepdims=True)
    j_r = lax.broadcasted_iota(jnp.int32, (32, EP), 0)
    e_l = lax.broadcasted_iota(jnp.int32, (32, EP), 1)
    covered = ((j_r >= bstart) & (e_l < E)).astype(jnp.int32)
    be = jnp.sum(covered, axis=1, keepdims=True) - 1
    j_c = lax.broadcasted_iota(jnp.int32, (32, 1), 0)
    be = jnp.where(j_c < nbtot, be, last_e)
    bmeta_ref[...] = jnp.where(j_c == 31, nbtot, be)


def _router_call(x, wr_p):
    return pl.pallas_call(
        _router_body,
        out_shape=(
            jax.ShapeDtypeStruct((T, 1), jnp.int32),
            jax.ShapeDtypeStruct((T, 1), jnp.float32),
            jax.ShapeDtypeStruct((32, 1), jnp.int32),
        ),
    )(x, wr_p)


TG = T // NS
ZG = PMAX // NS


def _dispatch_body(pos_hbm, gate_hbm, gpad_hbm, idxp_hbm,
                   posg_v, gatew_v, tid_v, zf_v, zi_v, gpw_v, ipw_v,
                   gate_sh, idx_sh):
    c = lax.axis_index("c")
    s = lax.axis_index("s")
    gb = s * TG
    zb = s * ZG
    pltpu.sync_copy(pos_hbm.at[pl.ds(gb, TG)], posg_v)

    @pl.when(c == 0)
    def _():
        pltpu.sync_copy(gate_hbm.at[pl.ds(gb, TG)], gatew_v)
        zeros16 = jnp.zeros((16,), jnp.float32)

        def _init(i, carry):
            zf_v[pl.ds(i * 16, 16)] = zeros16
            return carry

        lax.fori_loop(0, ZG // 16, _init, 0)
        pltpu.sync_copy(zf_v, gate_sh.at[pl.ds(zb, ZG)])
        plsc.subcore_barrier()
        pltpu.sync_copy(gatew_v, gate_sh.at[posg_v], add=True)
        plsc.subcore_barrier()
        pltpu.sync_copy(gate_sh.at[pl.ds(zb, ZG)], gpw_v)
        pltpu.sync_copy(gpw_v, gpad_hbm.at[pl.ds(zb, ZG)])

    @pl.when(c == 1)
    def _():
        izeros16 = jnp.zeros((16,), jnp.int32)
        lane = lax.broadcasted_iota(jnp.int32, (16,), 0)

        def _tid(i, carry):
            tid_v[pl.ds(i * 16, 16)] = lane + (gb + i * 16 + 1)
            return carry

        lax.fori_loop(0, TG // 16, _tid, 0)

        def _initi(i, carry):
            zi_v[pl.ds(i * 16, 16)] = izeros16
            return carry

        lax.fori_loop(0, ZG // 16, _initi, 0)
        pltpu.sync_copy(zi_v, idx_sh.at[pl.ds(zb, ZG)])
        plsc.subcore_barrier()
        pltpu.sync_copy(tid_v, idx_sh.at[posg_v], add=True)
        plsc.subcore_barrier()
        pltpu.sync_copy(idx_sh.at[pl.ds(zb, ZG)], ipw_v)
        pltpu.sync_copy(ipw_v, idxp_hbm.at[pl.ds(zb, ZG)])


def _ffn_body(bm_ref, idx_ref, x_ref, w1_ref, w2_ref, g_ref, out_ref,
              xg_ref):
    j = pl.program_id(0)
    nbtot = bm_ref[31]

    @pl.when(j < nbtot)
    def _():
        def _gather(i, carry):
            tok = jnp.maximum(idx_ref[j * BLK + i] - 1, 0)
            xg_ref[pl.ds(i, 1), :] = x_ref[pl.ds(tok, 1), :]
            return carry

        lax.fori_loop(0, BLK, _gather, 0, unroll=4)
        h = jax.nn.gelu(jnp.dot(xg_ref[...], w1_ref[0],
                                preferred_element_type=jnp.float32))
        o = jnp.dot(h, w2_ref[0], preferred_element_type=jnp.float32)
        out_ref[...] = o * g_ref[0]


def _ffn_call(bmeta, idxp, x, w1, w2, gpad3):
    grid_spec = pltpu.PrefetchScalarGridSpec(
        num_scalar_prefetch=2,
        grid=(NB,),
        in_specs=[
            pl.BlockSpec((T, D), lambda j, bm, ix: (0, 0)),
            pl.BlockSpec((1, D, F), lambda j, bm, ix: (bm[j], 0, 0)),
            pl.BlockSpec((1, F, D), lambda j, bm, ix: (bm[j], 0, 0)),
            pl.BlockSpec((1, BLK, 1),
                         lambda j, bm, ix: (jnp.minimum(j, bm[31] - 1), 0, 0)),
        ],
        out_specs=pl.BlockSpec(
            (BLK, D), lambda j, bm, ix: (jnp.minimum(j, bm[31] - 1), 0)),
        scratch_shapes=[pltpu.VMEM((BLK, D), jnp.float32)],
    )
    return pl.pallas_call(
        _ffn_body,
        grid_spec=grid_spec,
        out_shape=jax.ShapeDtypeStruct((PMAX, D), jnp.float32),
        compiler_params=pltpu.CompilerParams(
            dimension_semantics=("arbitrary",)),
    )(bmeta, idxp, x, w1, w2, gpad3)


def _combine_body(os_hbm, pos_hbm, out_hbm, posw_v, rows_v, sem):
    c = lax.axis_index("c")
    s = lax.axis_index("s")
    wid = s * NC + c
    base = wid * TPW
    pltpu.sync_copy(pos_hbm.at[pl.ds(base, TPW)], posw_v)
    pltpu.async_copy(os_hbm.at[posw_v], rows_v, sem).wait()
    pltpu.sync_copy(rows_v, out_hbm.at[pl.ds(base, TPW)])


@functools.cache
def _sc_kernels():
    mesh = plsc.VectorSubcoreMesh(core_axis_name="c", subcore_axis_name="s")
    dispatch = pl.kernel(
        _dispatch_body,
        mesh=mesh,
        out_type=[
            jax.ShapeDtypeStruct((PMAX,), jnp.float32),
            jax.ShapeDtypeStruct((PMAX,), jnp.int32),
        ],
        scratch_types=[
            pltpu.VMEM((TG,), jnp.int32),
            pltpu.VMEM((TG,), jnp.float32),
            pltpu.VMEM((TG,), jnp.int32),
            pltpu.VMEM((ZG,), jnp.float32),
            pltpu.VMEM((ZG,), jnp.int32),
            pltpu.VMEM((ZG,), jnp.float32),
            pltpu.VMEM((ZG,), jnp.int32),
            pltpu.VMEM_SHARED((PMAX,), jnp.float32),
            pltpu.VMEM_SHARED((PMAX,), jnp.int32),
        ],
        compiler_params=pltpu.CompilerParams(needs_layout_passes=False),
    )
    combine = pl.kernel(
        _combine_body,
        mesh=mesh,
        out_type=jax.ShapeDtypeStruct((T, D), jnp.float32),
        scratch_types=[
            pltpu.VMEM((TPW,), jnp.int32),
            pltpu.VMEM((TPW, D), jnp.float32),
            pltpu.SemaphoreType.DMA,
        ],
        compiler_params=pltpu.CompilerParams(needs_layout_passes=False),
    )
    return dispatch, combine


def kernel(x, w_router, w1, w2):
    wr_p = jnp.pad(w_router, ((0, 0), (0, EP - E)))
    pos2, gate2, bmeta2 = _router_call(x, wr_p)
    pos = pos2.reshape(T)
    gate = gate2.reshape(T)
    bmeta = bmeta2.reshape(32)
    dispatch, combine = _sc_kernels()
    gpad, idxp = dispatch(pos, gate)
    gpad3 = gpad.reshape(NB, BLK, 1)
    out_sorted = _ffn_call(bmeta, idxp, x, w1, w2, gpad3)
    return combine(out_sorted, pos)


# speedup vs baseline: 1.1117x; 1.0606x over previous
import functools

import jax
import jax.numpy as jnp
from jax import lax
from jax.experimental import pallas as pl
from jax.experimental.pallas import tpu as pltpu
from jax.experimental.pallas import tpu_sc as plsc

T, D, F, E = 2048, 768, 3072, 8
EP = 128
BLK = 128
NB = T // BLK + E
PMAX = NB * BLK
NC, NS = 2, 16
NW = NC * NS
RP = PMAX // NW
TPW = T // NW


def _router_body(x_ref, wr_ref, pos_ref, gate_ref, bmeta_ref):
    x = x_ref[...]
    logits = jnp.dot(x, wr_ref[...], preferred_element_type=jnp.float32)
    lane = lax.broadcasted_iota(jnp.int32, (T, EP), 1)
    valid = lane < E
    logits = jnp.where(valid, logits, jnp.float32(-1e30))
    m = jnp.max(logits, axis=1, keepdims=True)
    p = jnp.where(valid, jnp.exp(logits - m), 0.0)
    probs = p / jnp.sum(p, axis=1, keepdims=True)
    gate = jnp.max(probs, axis=1, keepdims=True)
    cand = jnp.where((probs >= gate) & valid, lane, EP)
    expert = jnp.min(cand, axis=1, keepdims=True)
    onehot = (lane == expert).astype(jnp.float32)
    counts = jnp.sum(onehot, axis=0, keepdims=True)
    r_i = lax.broadcasted_iota(jnp.int32, (T, T), 0)
    c_i = lax.broadcasted_iota(jnp.int32, (T, T), 1)
    ltri = (c_i < r_i).astype(jnp.float32)
    ranks_all = jnp.dot(ltri, onehot, preferred_element_type=jnp.float32)
    rank = jnp.sum(ranks_all * onehot, axis=1, keepdims=True)
    nb_e = (counts.astype(jnp.int32) + (BLK - 1)) // BLK
    e_r = lax.broadcasted_iota(jnp.int32, (EP, EP), 0)
    e_c = lax.broadcasted_iota(jnp.int32, (EP, EP), 1)
    ustrict = (e_r < e_c).astype(jnp.float32)
    nb8 = jnp.broadcast_to(nb_e.astype(jnp.float32), (8, EP))
    bstart = jnp.dot(nb8, ustrict,
                     preferred_element_type=jnp.float32)[0:1].astype(jnp.int32)
    pstart = bstart * BLK
    pos = (jnp.sum(onehot * pstart.astype(jnp.float32), axis=1, keepdims=True)
           + rank).astype(jnp.int32)
    pos_ref[...] = pos
    gate_ref[...] = gate
    nbtot = jnp.sum(nb_e, axis=1, keepdims=True)
    lane1 = lax.broadcasted_iota(jnp.int32, (1, EP), 1)
    last_e = jnp.max(jnp.where(nb_e > 0, lane1, 0), axis=1, keepdims=True)
    j_r = lax.broadcasted_iota(jnp.int32, (32, EP), 0)
    e_l = lax.broadcasted_iota(jnp.int32, (32, EP), 1)
    covered = ((j_r >= bstart) & (e_l < E)).astype(jnp.int32)
    be = jnp.sum(covered, axis=1, keepdims=True) - 1
    j_c = lax.broadcasted_iota(jnp.int32, (32, 1), 0)
    be = jnp.where(j_c < nbtot, be, last_e)
    bmeta_ref[...] = jnp.where(j_c == 31, nbtot, be)


def _router_call(x, wr_p):
    return pl.pallas_call(
        _router_body,
        out_shape=(
            jax.ShapeDtypeStruct((T, 1), jnp.int32),
            jax.ShapeDtypeStruct((T, 1), jnp.float32),
            jax.ShapeDtypeStruct((32, 1), jnp.int32),
        ),
    )(x, wr_p)


TG = T // NS
ZG = PMAX // NS


def _dispatch_body(pos_hbm, gate_hbm, gpad_hbm, idxp_hbm,
                   posg_v, gatew_v, tid_v, zf_v, zi_v, gpw_v, ipw_v,
                   gate_sh, idx_sh):
    c = lax.axis_index("c")
    s = lax.axis_index("s")
    gb = s * TG
    zb = s * ZG
    pltpu.sync_copy(pos_hbm.at[pl.ds(gb, TG)], posg_v)

    @pl.when(c == 0)
    def _():
        pltpu.sync_copy(gate_hbm.at[pl.ds(gb, TG)], gatew_v)
        zeros16 = jnp.zeros((16,), jnp.float32)

        def _init(i, carry):
            zf_v[pl.ds(i * 16, 16)] = zeros16
            return carry

        lax.fori_loop(0, ZG // 16, _init, 0)
        pltpu.sync_copy(zf_v, gate_sh.at[pl.ds(zb, ZG)])
        plsc.subcore_barrier()
        pltpu.sync_copy(gatew_v, gate_sh.at[posg_v], add=True)
        plsc.subcore_barrier()
        pltpu.sync_copy(gate_sh.at[pl.ds(zb, ZG)], gpw_v)
        pltpu.sync_copy(gpw_v, gpad_hbm.at[pl.ds(zb, ZG)])

    @pl.when(c == 1)
    def _():
        izeros16 = jnp.zeros((16,), jnp.int32)
        lane = lax.broadcasted_iota(jnp.int32, (16,), 0)

        def _tid(i, carry):
            tid_v[pl.ds(i * 16, 16)] = lane + (gb + i * 16 + 1)
            return carry

        lax.fori_loop(0, TG // 16, _tid, 0)

        def _initi(i, carry):
            zi_v[pl.ds(i * 16, 16)] = izeros16
            return carry

        lax.fori_loop(0, ZG // 16, _initi, 0)
        pltpu.sync_copy(zi_v, idx_sh.at[pl.ds(zb, ZG)])
        plsc.subcore_barrier()
        pltpu.sync_copy(tid_v, idx_sh.at[posg_v], add=True)
        plsc.subcore_barrier()
        pltpu.sync_copy(idx_sh.at[pl.ds(zb, ZG)], ipw_v)
        pltpu.sync_copy(ipw_v, idxp_hbm.at[pl.ds(zb, ZG)])


def _gather_block(idx_ref, x_ref, dst_ref, blk):
    for i in range(BLK):
        tok = jnp.maximum(idx_ref[blk * BLK + i] - 1, 0)
        dst_ref[pl.ds(i, 1), :] = x_ref[pl.ds(tok, 1), :]


def _ffn_body(bm_ref, idx_ref, x_ref, w1_ref, w2_ref, g_ref, out_ref,
              xga_ref, xgb_ref):
    j = pl.program_id(0)
    nbtot = bm_ref[31]
    nxt = jnp.minimum(j + 1, nbtot - 1)

    def _compute(src_ref):
        h = jax.nn.gelu(jnp.dot(src_ref[...], w1_ref[0],
                                preferred_element_type=jnp.float32))
        o = jnp.dot(h, w2_ref[0], preferred_element_type=jnp.float32)
        out_ref[...] = o * g_ref[0]

    @pl.when(j == 0)
    def _():
        _gather_block(idx_ref, x_ref, xga_ref, 0)

    even = lax.rem(j, 2) == 0

    @pl.when((j < nbtot) & even)
    def _():
        _gather_block(idx_ref, x_ref, xgb_ref, nxt)
        _compute(xga_ref)

    @pl.when((j < nbtot) & jnp.logical_not(even))
    def _():
        _gather_block(idx_ref, x_ref, xga_ref, nxt)
        _compute(xgb_ref)


def _ffn_call(bmeta, idxp, x, w1, w2, gpad3):
    grid_spec = pltpu.PrefetchScalarGridSpec(
        num_scalar_prefetch=2,
        grid=(NB,),
        in_specs=[
            pl.BlockSpec((T, D), lambda j, bm, ix: (0, 0)),
            pl.BlockSpec((1, D, F), lambda j, bm, ix: (bm[j], 0, 0)),
            pl.BlockSpec((1, F, D), lambda j, bm, ix: (bm[j], 0, 0)),
            pl.BlockSpec((1, BLK, 1),
                         lambda j, bm, ix: (jnp.minimum(j, bm[31] - 1), 0, 0)),
        ],
        out_specs=pl.BlockSpec(
            (BLK, D), lambda j, bm, ix: (jnp.minimum(j, bm[31] - 1), 0)),
        scratch_shapes=[pltpu.VMEM((BLK, D), jnp.float32),
                        pltpu.VMEM((BLK, D), jnp.float32)],
    )
    return pl.pallas_call(
        _ffn_body,
        grid_spec=grid_spec,
        out_shape=jax.ShapeDtypeStruct((PMAX, D), jnp.float32),
        compiler_params=pltpu.CompilerParams(
            dimension_semantics=("arbitrary",)),
    )(bmeta, idxp, x, w1, w2, gpad3)


def _combine_body(os_hbm, pos_hbm, out_hbm, posw_v, rows_v, sem):
    c = lax.axis_index("c")
    s = lax.axis_index("s")
    wid = s * NC + c
    base = wid * TPW
    pltpu.sync_copy(pos_hbm.at[pl.ds(base, TPW)], posw_v)
    pltpu.async_copy(os_hbm.at[posw_v], rows_v, sem).wait()
    pltpu.sync_copy(rows_v, out_hbm.at[pl.ds(base, TPW)])


@functools.cache
def _sc_kernels():
    mesh = plsc.VectorSubcoreMesh(core_axis_name="c", subcore_axis_name="s")
    dispatch = pl.kernel(
        _dispatch_body,
        mesh=mesh,
        out_type=[
            jax.ShapeDtypeStruct((PMAX,), jnp.float32),
            jax.ShapeDtypeStruct((PMAX,), jnp.int32),
        ],
        scratch_types=[
            pltpu.VMEM((TG,), jnp.int32),
            pltpu.VMEM((TG,), jnp.float32),
            pltpu.VMEM((TG,), jnp.int32),
            pltpu.VMEM((ZG,), jnp.float32),
            pltpu.VMEM((ZG,), jnp.int32),
            pltpu.VMEM((ZG,), jnp.float32),
            pltpu.VMEM((ZG,), jnp.int32),
            pltpu.VMEM_SHARED((PMAX,), jnp.float32),
            pltpu.VMEM_SHARED((PMAX,), jnp.int32),
        ],
        compiler_params=pltpu.CompilerParams(needs_layout_passes=False),
    )
    combine = pl.kernel(
        _combine_body,
        mesh=mesh,
        out_type=jax.ShapeDtypeStruct((T, D), jnp.float32),
        scratch_types=[
            pltpu.VMEM((TPW,), jnp.int32),
            pltpu.VMEM((TPW, D), jnp.float32),
            pltpu.SemaphoreType.DMA,
        ],
        compiler_params=pltpu.CompilerParams(needs_layout_passes=False),
    )
    return dispatch, combine


def kernel(x, w_router, w1, w2):
    wr_p = jnp.pad(w_router, ((0, 0), (0, EP - E)))
    pos2, gate2, bmeta2 = _router_call(x, wr_p)
    pos = pos2.reshape(T)
    gate = gate2.reshape(T)
    bmeta = bmeta2.reshape(32)
    dispatch, combine = _sc_kernels()
    gpad, idxp = dispatch(pos, gate)
    gpad3 = gpad.reshape(NB, BLK, 1)
    out_sorted = _ffn_call(bmeta, idxp, x, w1, w2, gpad3)
    return combine(out_sorted, pos)
